# Initial kernel scaffold; baseline (speedup 1.0000x reference)
import jax
import jax.numpy as jnp
from jax.experimental import pallas as pl


def kernel(node_feat, adj_matrix, W, attn_l, attn_r):
    raise NotImplementedError("write your pallas kernel here")


# speedup vs baseline: 1.2780x; 1.2780x over previous
import jax
import jax.numpy as jnp
from jax.experimental import pallas as pl


def _gat_fused(nf_ref, adj_ref, w_ref, al_ref, ar_ref, out_ref, att_ref):
    H, D = al_ref.shape
    feat = jnp.dot(nf_ref[0], w_ref[...], preferred_element_type=jnp.float32)
    mask = adj_ref[0] > 0
    for h in range(H):
        feat_h = feat[:, h * D:(h + 1) * D]
        al_h = al_ref[h, :].reshape(1, D)
        ar_h = ar_ref[h, :].reshape(1, D)
        el = jax.lax.dot_general(feat_h, al_h, (((1,), (1,)), ((), ())),
                                 preferred_element_type=jnp.float32)
        er = jax.lax.dot_general(ar_h, feat_h, (((1,), (1,)), ((), ())),
                                 preferred_element_type=jnp.float32)
        s = el + er
        s = jnp.where(s >= 0.0, s, 0.2 * s)
        neg = jnp.where(mask, s, -1e30)
        m = jnp.max(neg, axis=0, keepdims=True)
        ex = jnp.where(mask, jnp.exp(neg - m), 0.0)
        denom = jnp.sum(ex, axis=0, keepdims=True)
        a = ex / jnp.maximum(denom, 1e-20)
        att_ref[0, h] = a
        out_ref[0, :, h * D:(h + 1) * D] = jnp.dot(
            a, feat_h, preferred_element_type=jnp.float32)


def kernel(node_feat, adj_matrix, W, attn_l, attn_r):
    B, N, in_dim = node_feat.shape
    H, D = attn_l.shape[1], attn_l.shape[2]
    out, att = pl.pallas_call(
        _gat_fused,
        grid=(B,),
        in_specs=[
            pl.BlockSpec((1, N, in_dim), lambda b: (b, 0, 0)),
            pl.BlockSpec((1, N, N), lambda b: (b, 0, 0)),
            pl.BlockSpec((in_dim, H * D), lambda b: (0, 0)),
            pl.BlockSpec((H, D), lambda b: (0, 0)),
            pl.BlockSpec((H, D), lambda b: (0, 0)),
        ],
        out_specs=[
            pl.BlockSpec((1, N, H * D), lambda b: (b, 0, 0)),
            pl.BlockSpec((1, H, N, N), lambda b: (b, 0, 0, 0)),
        ],
        out_shape=[
            jax.ShapeDtypeStruct((B, N, H * D), jnp.float32),
            jax.ShapeDtypeStruct((B, H, N, N), jnp.float32),
        ],
    )(node_feat, adj_matrix, W,
      attn_l.reshape(H, D), attn_r.reshape(H, D))
    attention = jnp.transpose(att, (0, 2, 3, 1))
    return out, attention
